# baseline (speedup 1.0000x reference)
import functools

import jax
import jax.numpy as jnp
from jax import lax
from jax.experimental import pallas as pl
from jax.experimental.pallas import tpu as pltpu
from jax.experimental.pallas import tpu_sc as plsc

NC = 2
NS = 16
NW = NC * NS
LANES = 16
SLOTS = 126


def _prepack_body(off_c, off_n, obs_ref, coords_ref, n_ref, out_ref):
    cb = out_ref.shape[0]
    o = obs_ref[...].reshape(cb, -1)
    c = coords_ref[...] * 2 + off_c
    out_ref[...] = jnp.concatenate(
        [c[:, 0:1], c[:, 0:1] + 1, c[:, 1:2], c[:, 1:2] + 1,
         o, n_ref[...] + off_n, o[:, 0:2]], axis=1)


def _sc_body(C, comb_hbm, idx_hbm, out_hbm, obsidx, outbuf, sem):
    wid = lax.axis_index("s") * NC + lax.axis_index("c")
    B = out_hbm.shape[0]
    rows_per = B // NW
    nch = rows_per // C

    @pl.loop(0, nch)
    def _chunk(g):
        r0 = wid * rows_per + g * C
        pltpu.sync_copy(idx_hbm.at[pl.ds(r0, C)], obsidx)
        cps = [pltpu.async_copy(comb_hbm.at[obsidx.at[i]],
                                outbuf.at[i], sem)
               for i in range(C)]
        for cp in cps:
            cp.wait()
        pltpu.sync_copy(outbuf, out_hbm.at[pl.ds(r0, C)])


def kernel(coords, obses, n_completed, coord_table, field_table,
           completed_table):
    B = coords.shape[0]
    coords = coords.astype(jnp.int32)
    obses = obses.astype(jnp.int32)
    n_completed = n_completed.astype(jnp.int32)
    fdim = field_table.shape[1]
    off_n = field_table.shape[0]
    off_c = off_n + completed_table.shape[0]
    comb = jnp.concatenate(
        [field_table, completed_table, coord_table.reshape(-1, fdim)], axis=0)

    o = obses.reshape(B, -1)
    c2 = coords * 2 + off_c
    idxrows = jnp.concatenate(
        [c2[:, 0:1], c2[:, 0:1] + 1, c2[:, 1:2], c2[:, 1:2] + 1,
         o, n_completed + off_n, o[:, 0:2]], axis=1)
    idxrows = lax.optimization_barrier(idxrows)

    C = 32
    mesh = plsc.VectorSubcoreMesh(core_axis_name="c", subcore_axis_name="s")
    out = pl.kernel(
        functools.partial(_sc_body, C),
        out_type=jax.ShapeDtypeStruct((B, 128, fdim), jnp.float32),
        mesh=mesh,
        compiler_params=pltpu.CompilerParams(
            use_tc_tiling_on_sc=False,
            needs_layout_passes=False,
        ),
        scratch_types=[
            pltpu.VMEM((C, 128), jnp.int32),
            pltpu.VMEM((C, 128, fdim), jnp.float32),
            pltpu.SemaphoreType.DMA,
        ],
    )(comb, idxrows)
    return out[:, :SLOTS, :].reshape(B, SLOTS * fdim)


# speedup vs baseline: 1.3022x; 1.3022x over previous
import functools

import jax
import jax.numpy as jnp
from jax import lax
from jax.experimental import pallas as pl
from jax.experimental.pallas import tpu as pltpu
from jax.experimental.pallas import tpu_sc as plsc

NC = 2
NS = 16
NW = NC * NS
LANES = 16
SLOTS = 126


def _sc_body(C,
             coord2_hbm, field_hbm, comp_hbm, cflat_hbm, obs_hbm, n_hbm,
             out_hbm, obsidx, cidx, craw, nraw, outbuf, sem):
    wid = lax.axis_index("s") * NC + lax.axis_index("c")
    B = out_hbm.shape[0]
    rows_per = B // NW
    nch = rows_per // C
    iota = lax.broadcasted_iota(jnp.int32, (LANES,), 0)

    @pl.loop(0, nch)
    def _chunk(g):
        r0 = wid * rows_per + g * C
        pltpu.sync_copy(obs_hbm.at[pl.ds(r0, C)], obsidx)
        pltpu.sync_copy(cflat_hbm.at[pl.ds(2 * r0, 2 * C)], craw)
        pltpu.sync_copy(n_hbm.at[pl.ds(r0, C)], nraw)
        for k in range((2 * C) // LANES):
            p = iota + (k * LANES)
            c = craw[pl.ds(k * LANES, LANES)]
            pos = jnp.right_shift(p, 1) * 16 + jnp.bitwise_and(p, 1) * 2
            plsc.store_scatter(cidx, [pos], c * 2)
            plsc.store_scatter(cidx, [pos + 1], c * 2 + 1)
        for k in range(C // LANES):
            pos = (iota + (k * LANES)) * 16 + 8
            n = nraw[pl.ds(k * LANES, LANES)]
            plsc.store_scatter(cidx, [pos], n)
        cps = []
        for i in range(C):
            cps.append(pltpu.async_copy(
                coord2_hbm.at[cidx.at[pl.ds(16 * i, 4)]],
                outbuf.at[i, pl.ds(0, 4)], sem))
            cps.append(pltpu.async_copy(
                field_hbm.at[obsidx.at[i]],
                outbuf.at[i, pl.ds(4, 121)], sem))
            cps.append(pltpu.async_copy(
                comp_hbm.at[cidx.at[pl.ds(16 * i + 8, 1)]],
                outbuf.at[i, pl.ds(SLOTS - 1, 1)], sem))
        for cp in cps:
            cp.wait()
        pltpu.sync_copy(outbuf, out_hbm.at[pl.ds(r0, C)])


def kernel(coords, obses, n_completed, coord_table, field_table,
           completed_table):
    B = coords.shape[0]
    coords = coords.astype(jnp.int32)
    obses = obses.astype(jnp.int32)
    n_completed = n_completed.astype(jnp.int32)
    fdim = field_table.shape[1]
    coord2 = coord_table.reshape(-1, fdim)
    obs2 = obses.reshape(B, -1)
    cflat = coords.reshape(-1)
    nflat = n_completed.reshape(-1)

    C = 32
    mesh = plsc.VectorSubcoreMesh(core_axis_name="c", subcore_axis_name="s")
    out = pl.kernel(
        functools.partial(_sc_body, C),
        out_type=jax.ShapeDtypeStruct((B, SLOTS, fdim), jnp.float32),
        mesh=mesh,
        compiler_params=pltpu.CompilerParams(
            use_tc_tiling_on_sc=False,
            needs_layout_passes=False,
        ),
        scratch_types=[
            pltpu.VMEM((C, 121), jnp.int32),
            pltpu.VMEM((16 * C,), jnp.int32),
            pltpu.VMEM((2 * C,), jnp.int32),
            pltpu.VMEM((C,), jnp.int32),
            pltpu.VMEM((C, SLOTS, fdim), jnp.float32),
            pltpu.SemaphoreType.DMA,
        ],
    )(coord2, field_table, completed_table, cflat, obs2, nflat)
    return out.reshape(B, SLOTS * fdim)
